# baseline (speedup 1.0000x reference)
import jax
import jax.numpy as jnp
from jax import lax
from jax.experimental import pallas as pl
from jax.experimental.pallas import tpu as pltpu
from jax.experimental.pallas import tpu_sc as plsc

DIM = 64
NCTX = 5
NLANE = 16
NC = 2
NS = 16
NW = NC * NS

CHUNK = 128
CTX_CHUNK = CHUNK * NCTX
GROUP = 16
NGROUP = CHUNK // GROUP


def _sc_body(tgt_idx_hbm, ctx_idx_hbm, tgt_tab_hbm, ctx_tab_hbm, out_hbm,
             tgt_idx_v, ctx_idx_v, tgt_rows_v, ctx_rows_v, out_v, sem):
    batch = out_hbm.shape[0] // NCTX
    b_per_w = batch // NW
    nchunk = b_per_w // CHUNK
    wid = lax.axis_index("s") * NC + lax.axis_index("c")
    lane = lax.iota(jnp.int32, NLANE)

    def chunk_body(g, carry):
        base = wid * b_per_w + g * CHUNK
        pltpu.sync_copy(tgt_idx_hbm.at[pl.ds(base, CHUNK)], tgt_idx_v)
        pltpu.sync_copy(ctx_idx_hbm.at[pl.ds(base * NCTX, CTX_CHUNK)],
                        ctx_idx_v)
        cps = [pltpu.async_copy(tgt_tab_hbm.at[tgt_idx_v], tgt_rows_v, sem)]
        for j in range(NCTX):
            cps.append(pltpu.async_copy(
                ctx_tab_hbm.at[ctx_idx_v.at[pl.ds(j * CHUNK, CHUNK)]],
                ctx_rows_v.at[pl.ds(j * CHUNK, CHUNK)], sem))
        for cp in cps:
            cp.wait()

        def group_body(bg, inner_carry):
            b_base = bg * GROUP
            vecs = [jnp.zeros((NLANE,), jnp.float32) for _ in range(NCTX)]
            for b0 in range(GROUP):
                b = b_base + b0
                t0 = tgt_rows_v[b, pl.ds(0, NLANE)]
                t1 = tgt_rows_v[b, pl.ds(NLANE, NLANE)]
                t2 = tgt_rows_v[b, pl.ds(2 * NLANE, NLANE)]
                t3 = tgt_rows_v[b, pl.ds(3 * NLANE, NLANE)]
                for c in range(NCTX):
                    r = b * NCTX + c
                    acc = ctx_rows_v[r, pl.ds(0, NLANE)] * t0
                    acc = acc + ctx_rows_v[r, pl.ds(NLANE, NLANE)] * t1
                    acc = acc + ctx_rows_v[r, pl.ds(2 * NLANE, NLANE)] * t2
                    acc = acc + ctx_rows_v[r, pl.ds(3 * NLANE, NLANE)] * t3
                    s = jnp.sum(acc)
                    p = b0 * NCTX + c
                    j, l = p // NLANE, p % NLANE
                    vecs[j] = jnp.where(lane == l, s, vecs[j])
            for j in range(NCTX):
                out_v[pl.ds(b_base * NCTX + j * NLANE, NLANE)] = vecs[j]
            return inner_carry

        lax.fori_loop(0, NGROUP, group_body, 0)
        pltpu.sync_copy(out_v, out_hbm.at[pl.ds(base * NCTX, CTX_CHUNK)])
        return carry

    lax.fori_loop(0, nchunk, chunk_body, 0)


def _make_call(batch):
    mesh = plsc.VectorSubcoreMesh(core_axis_name="c", subcore_axis_name="s")
    return pl.kernel(
        _sc_body,
        mesh=mesh,
        out_type=jax.ShapeDtypeStruct((batch * NCTX,), jnp.float32),
        scratch_types=[
            pltpu.VMEM((CHUNK,), jnp.int32),
            pltpu.VMEM((CTX_CHUNK,), jnp.int32),
            pltpu.VMEM((CHUNK, DIM), jnp.float32),
            pltpu.VMEM((CTX_CHUNK, DIM), jnp.float32),
            pltpu.VMEM((CTX_CHUNK,), jnp.float32),
            pltpu.SemaphoreType.DMA,
        ],
        compiler_params=pltpu.CompilerParams(
            needs_layout_passes=False, use_tc_tiling_on_sc=False),
    )


@jax.jit
def kernel(target, context, target_table, context_table):
    batch = target.shape[0]
    tgt_flat = target.reshape(batch)
    ctx_flat = context.reshape(batch * NCTX)
    out_flat = _make_call(batch)(tgt_flat, ctx_flat,
                                 target_table, context_table)
    return out_flat.reshape(batch, NCTX)


# speedup vs baseline: 1.4725x; 1.4725x over previous
import jax
import jax.numpy as jnp
from jax import lax
from jax.experimental import pallas as pl
from jax.experimental.pallas import tpu as pltpu
from jax.experimental.pallas import tpu_sc as plsc

DIM = 64
NCTX = 5
NLANE = 16
NC = 2
NS = 16
HALF = DIM // NC
B = 16384
NLOOK = B * (NCTX + 1)
DUMP = NLOOK
SLOTS = NLOOK + 1

RNG = 62464
WINR = 1024
NWIN = RNG // WINR
SUPW = 8
NSUP = 8
TAILR = 512
FINR = 64

CAP_T, CAP_C = 2048, 7168
SCAP_T, SCAP_C = 768, 1536
GCAP_T, GCAP_C = 256, 512
CCHUNK = 64
VOCABF = NS * RNG + TAILR


def _filter(src_r, src_p, cnt, dst_r, dst_p, lo, width, cap):
    lane = lax.iota(jnp.int32, NLANE)

    def body(g, cur):
        rl = src_r[pl.ds(g * NLANE, NLANE)] - lo
        p = src_p[pl.ds(g * NLANE, NLANE)]
        m = (rl >= 0) & (rl < width)
        plsc.store_compressed(dst_r.at[pl.ds(cur, NLANE)], rl, mask=m)
        plsc.store_compressed(dst_p.at[pl.ds(cur, NLANE)], p, mask=m)
        cnt_vec = plsc.all_reduce_population_count(m)
        return jnp.minimum(cur + cnt_vec[0], cap)

    ngroups = lax.div(cnt + NLANE - 1, NLANE)
    out = lax.fori_loop(0, ngroups, body, 0)
    dst_r[pl.ds(out, NLANE)] = jnp.zeros((NLANE,), jnp.int32)
    dst_p[pl.ds(out, NLANE)] = jnp.full((NLANE,), DUMP, jnp.int32)
    del lane
    return out


def _extract(gr, gp, cnt, win, rowbuf, drain_v, stage_hbm, c, sem):
    lane = lax.iota(jnp.int32, NLANE)

    def body(g, carry):
        rlv = gr[pl.ds(g * NLANE, NLANE)]
        plv = gp[pl.ds(g * NLANE, NLANE)]
        for l in range(NLANE):
            rs = jnp.full((NLANE,), rlv[l], jnp.int32)
            rowbuf[l, pl.ds(0, NLANE)] = plsc.load_gather(win, [lane, rs])
            rowbuf[l, pl.ds(NLANE, NLANE)] = plsc.load_gather(
                win, [lane + NLANE, rs])
            pltpu.async_copy(
                rowbuf.at[l],
                stage_hbm.at[pl.ds((c * SLOTS + plv[l]) * HALF, HALF)],
                sem)
        pltpu.make_async_copy(
            stage_hbm.at[pl.ds(0, NLANE * HALF)], drain_v, sem).wait()
        return carry

    lax.fori_loop(0, lax.div(cnt + NLANE - 1, NLANE), body, 0)


def _sc_body(idx_all_hbm, tgt_tab_hbm, ctx_tab_hbm, tail_t_hbm, tail_c_hbm,
             out_hbm, stage_hbm,
             idx_v, wl_tr, wl_tp, wl_cr, wl_cp, swl_r, swl_p, gwl_r, gwl_p,
             win_t, win_c, win_ft, win_fc, rowbuf, drain_v, tbuf, cbuf,
             out_v, sem):
    c = lax.axis_index("c")
    s = lax.axis_index("s")
    lane = lax.iota(jnp.int32, NLANE)
    r0 = s * RNG

    def scan_chunk(ci, wr, wp, cap, cur0):
        pltpu.sync_copy(idx_all_hbm.at[pl.ds(ci * 8192, 8192)], idx_v)

        def body(v, cur):
            rl = idx_v[pl.ds(v * NLANE, NLANE)] - r0
            width = jnp.where(s == NS - 1, RNG + TAILR + FINR, RNG)
            m = (rl >= 0) & (rl < width)
            plsc.store_compressed(wr.at[pl.ds(cur, NLANE)], rl, mask=m)
            p = ci * 8192 + v * NLANE + lane
            plsc.store_compressed(wp.at[pl.ds(cur, NLANE)], p, mask=m)
            cnt_vec = plsc.all_reduce_population_count(m)
            return jnp.minimum(cur + cnt_vec[0], cap)

        return lax.fori_loop(0, 8192 // NLANE, body, cur0)

    cnt_t = 0
    for ci in range(2):
        cnt_t = scan_chunk(ci, wl_tr, wl_tp, CAP_T, cnt_t)
    cnt_c = 0
    for ci in range(2, 12):
        cnt_c = scan_chunk(ci, wl_cr, wl_cp, CAP_C, cnt_c)
    wl_tr[pl.ds(cnt_t, NLANE)] = jnp.zeros((NLANE,), jnp.int32)
    wl_tp[pl.ds(cnt_t, NLANE)] = jnp.full((NLANE,), DUMP, jnp.int32)
    wl_cr[pl.ds(cnt_c, NLANE)] = jnp.zeros((NLANE,), jnp.int32)
    wl_cp[pl.ds(cnt_c, NLANE)] = jnp.full((NLANE,), DUMP, jnp.int32)

    def super_body(sp, carry):
        lo_sp = sp * (SUPW * WINR)
        nw = jnp.where(sp == NSUP - 1, NWIN - (NSUP - 1) * SUPW, SUPW)
        sc_t = _filter(wl_tr, wl_tp, cnt_t, swl_r, swl_p, lo_sp,
                       nw * WINR, SCAP_T)
        sc_c = _filter(wl_cr, wl_cp, cnt_c,
                       swl_r.at[pl.ds(SCAP_T + NLANE, SCAP_C + NLANE)],
                       swl_p.at[pl.ds(SCAP_T + NLANE, SCAP_C + NLANE)],
                       lo_sp, nw * WINR, SCAP_C)

        def win_body(w, carry2):
            rw = r0 + lo_sp + w * WINR
            pltpu.sync_copy(
                tgt_tab_hbm.at[pl.ds(c * HALF, HALF), pl.ds(rw, WINR)],
                win_t)
            pltpu.sync_copy(
                ctx_tab_hbm.at[pl.ds(c * HALF, HALF), pl.ds(rw, WINR)],
                win_c)
            g_t = _filter(swl_r, swl_p, sc_t, gwl_r, gwl_p,
                          w * WINR, WINR, GCAP_T)
            _extract(gwl_r, gwl_p, g_t, win_t, rowbuf, drain_v,
                     stage_hbm, c, sem)
            g_c = _filter(swl_r.at[pl.ds(SCAP_T + NLANE, SCAP_C + NLANE)],
                          swl_p.at[pl.ds(SCAP_T + NLANE, SCAP_C + NLANE)],
                          sc_c,
                          gwl_r.at[pl.ds(GCAP_T + NLANE, GCAP_C + NLANE)],
                          gwl_p.at[pl.ds(GCAP_T + NLANE, GCAP_C + NLANE)],
                          w * WINR, WINR, GCAP_C)
            _extract(gwl_r.at[pl.ds(GCAP_T + NLANE, GCAP_C + NLANE)],
                     gwl_p.at[pl.ds(GCAP_T + NLANE, GCAP_C + NLANE)],
                     g_c, win_c, rowbuf, drain_v, stage_hbm, c, sem)
            return carry2

        lax.fori_loop(0, nw, win_body, 0)
        return carry

    lax.fori_loop(0, NSUP, super_body, 0)

    @pl.when(s == NS - 1)
    def _():
        rw = (NS - 1) * RNG + NWIN * WINR
        pltpu.sync_copy(
            tgt_tab_hbm.at[pl.ds(c * HALF, HALF), pl.ds(rw, TAILR)],
            win_t.at[:, pl.ds(0, TAILR)])
        pltpu.sync_copy(
            ctx_tab_hbm.at[pl.ds(c * HALF, HALF), pl.ds(rw, TAILR)],
            win_c.at[:, pl.ds(0, TAILR)])
        g_t = _filter(wl_tr, wl_tp, cnt_t, gwl_r, gwl_p,
                      NWIN * WINR, TAILR, GCAP_T)
        _extract(gwl_r, gwl_p, g_t, win_t, rowbuf, drain_v,
                 stage_hbm, c, sem)
        g_c = _filter(wl_cr, wl_cp, cnt_c,
                      gwl_r.at[pl.ds(GCAP_T + NLANE, GCAP_C + NLANE)],
                      gwl_p.at[pl.ds(GCAP_T + NLANE, GCAP_C + NLANE)],
                      NWIN * WINR, TAILR, GCAP_C)
        _extract(gwl_r.at[pl.ds(GCAP_T + NLANE, GCAP_C + NLANE)],
                 gwl_p.at[pl.ds(GCAP_T + NLANE, GCAP_C + NLANE)],
                 g_c, win_c, rowbuf, drain_v, stage_hbm, c, sem)
        pltpu.sync_copy(tail_t_hbm.at[pl.ds(c * HALF, HALF)], win_ft)
        pltpu.sync_copy(tail_c_hbm.at[pl.ds(c * HALF, HALF)], win_fc)
        f_t = _filter(wl_tr, wl_tp, cnt_t, gwl_r, gwl_p,
                      NWIN * WINR + TAILR, FINR, GCAP_T)
        _extract(gwl_r, gwl_p, f_t, win_ft, rowbuf, drain_v,
                 stage_hbm, c, sem)
        f_c = _filter(wl_cr, wl_cp, cnt_c,
                      gwl_r.at[pl.ds(GCAP_T + NLANE, GCAP_C + NLANE)],
                      gwl_p.at[pl.ds(GCAP_T + NLANE, GCAP_C + NLANE)],
                      NWIN * WINR + TAILR, FINR, GCAP_C)
        _extract(gwl_r.at[pl.ds(GCAP_T + NLANE, GCAP_C + NLANE)],
                 gwl_p.at[pl.ds(GCAP_T + NLANE, GCAP_C + NLANE)],
                 f_c, win_fc, rowbuf, drain_v, stage_hbm, c, sem)

    plsc.subcore_barrier()

    def sub_body(sub, carry):
        bbase = s * 1024 + sub * CCHUNK
        pltpu.sync_copy(
            stage_hbm.at[pl.ds((c * SLOTS + bbase) * HALF, CCHUNK * HALF)],
            tbuf)
        pltpu.sync_copy(
            stage_hbm.at[pl.ds((c * SLOTS + B + bbase * NCTX) * HALF,
                               CCHUNK * NCTX * HALF)],
            cbuf)

        def b_body(b0, carry2):
            t0 = tbuf[pl.ds(b0 * HALF, NLANE)]
            t1 = tbuf[pl.ds(b0 * HALF + NLANE, NLANE)]
            vals = jnp.zeros((NLANE,), jnp.float32)
            for cc in range(NCTX):
                coff = (b0 * NCTX + cc) * HALF
                acc = cbuf[pl.ds(coff, NLANE)] * t0
                acc = acc + cbuf[pl.ds(coff + NLANE, NLANE)] * t1
                vals = jnp.where(lane == cc, jnp.sum(acc), vals)
            cur = out_v[pl.ds(b0 * NCTX, NLANE)]
            out_v[pl.ds(b0 * NCTX, NLANE)] = jnp.where(lane < NCTX, vals,
                                                       cur)
            return carry2

        lax.fori_loop(0, CCHUNK, b_body, 0)
        pltpu.sync_copy(
            out_v.at[pl.ds(0, CCHUNK * NCTX)],
            out_hbm.at[pl.ds((c * NS * 1024 + bbase) * NCTX,
                             CCHUNK * NCTX)])
        return carry

    lax.fori_loop(0, 1024 // CCHUNK, sub_body, 0)


def _make_call(batch):
    mesh = plsc.VectorSubcoreMesh(core_axis_name="c", subcore_axis_name="s")
    return pl.kernel(
        _sc_body,
        mesh=mesh,
        out_type=(
            jax.ShapeDtypeStruct((NC * batch * NCTX,), jnp.float32),
            jax.ShapeDtypeStruct((NC * SLOTS * HALF,), jnp.float32),
        ),
        scratch_types=[
            pltpu.VMEM((8192,), jnp.int32),
            pltpu.VMEM((CAP_T + NLANE,), jnp.int32),
            pltpu.VMEM((CAP_T + NLANE,), jnp.int32),
            pltpu.VMEM((CAP_C + NLANE,), jnp.int32),
            pltpu.VMEM((CAP_C + NLANE,), jnp.int32),
            pltpu.VMEM((SCAP_T + SCAP_C + 2 * NLANE,), jnp.int32),
            pltpu.VMEM((SCAP_T + SCAP_C + 2 * NLANE,), jnp.int32),
            pltpu.VMEM((GCAP_T + GCAP_C + 2 * NLANE,), jnp.int32),
            pltpu.VMEM((GCAP_T + GCAP_C + 2 * NLANE,), jnp.int32),
            pltpu.VMEM((HALF, WINR), jnp.float32),
            pltpu.VMEM((HALF, WINR), jnp.float32),
            pltpu.VMEM((HALF, FINR), jnp.float32),
            pltpu.VMEM((HALF, FINR), jnp.float32),
            pltpu.VMEM((NLANE, DIM // 2), jnp.float32),
            pltpu.VMEM((NLANE * HALF,), jnp.float32),
            pltpu.VMEM((CCHUNK * HALF,), jnp.float32),
            pltpu.VMEM((CCHUNK * NCTX * HALF,), jnp.float32),
            pltpu.VMEM((CCHUNK * NCTX + NLANE,), jnp.float32),
            pltpu.SemaphoreType.DMA,
        ],
        compiler_params=pltpu.CompilerParams(needs_layout_passes=False),
    )


@jax.jit
def kernel(target, context, target_table, context_table):
    batch = target.shape[0]
    idx_all = jnp.concatenate(
        [target.reshape(batch), context.reshape(batch * NCTX)])
    tail_t = target_table[VOCABF:].T
    tail_c = context_table[VOCABF:].T
    out, _ = _make_call(batch)(idx_all, target_table.T, context_table.T,
                               tail_t, tail_c)
    out2 = out.reshape(NC, batch * NCTX)
    return (out2[0] + out2[1]).reshape(batch, NCTX)
